# baseline (speedup 1.0000x reference)
import functools

import jax
import jax.numpy as jnp
from jax import lax
from jax.experimental import pallas as pl
from jax.experimental.pallas import tpu as pltpu
from jax.experimental.pallas import tpu_sc as plsc

D_MODEL = 2048
N_ROWS = 8192
NC, NS, L = 2, 16, 16
NW = NC * NS
RPW = N_ROWS // NW
K = 8
NCHUNK = RPW // K
CPR = D_MODEL // L


def _sc_body(x_hbm, inds_hbm, pe_hbm, out_hbm, idx_v, pe_v, x_v, sem):
    wid = lax.axis_index("s") * NC + lax.axis_index("c")
    base = wid * RPW
    pltpu.sync_copy(inds_hbm.at[pl.ds(base, RPW)], idx_v)

    def chunk(i, carry):
        rbase = base + i * K
        pltpu.sync_copy(x_hbm.at[pl.ds(rbase, K)], x_v)
        pltpu.async_copy(pe_hbm.at[idx_v.at[pl.ds(i * K, K)]], pe_v, sem).wait()

        def add_row(r, c2):
            def add_col(c, c3):
                x_v[r, pl.ds(c * L, L)] = (
                    x_v[r, pl.ds(c * L, L)] + pe_v[r, pl.ds(c * L, L)]
                )
                return c3
            return lax.fori_loop(0, CPR, add_col, c2)

        lax.fori_loop(0, K, add_row, 0)
        pltpu.sync_copy(x_v, out_hbm.at[pl.ds(rbase, K)])
        return carry

    lax.fori_loop(0, NCHUNK, chunk, 0)


_mesh = plsc.VectorSubcoreMesh(core_axis_name="c", subcore_axis_name="s")

_pe_add = functools.partial(
    pl.kernel,
    out_type=jax.ShapeDtypeStruct((N_ROWS, D_MODEL), jnp.float32),
    mesh=_mesh,
    scratch_types=[
        pltpu.VMEM((RPW,), jnp.int32),
        pltpu.VMEM((K, D_MODEL), jnp.float32),
        pltpu.VMEM((K, D_MODEL), jnp.float32),
        pltpu.SemaphoreType.DMA,
    ],
)(_sc_body)


def kernel(x, inds, pe):
    b, s, d = x.shape
    out = _pe_add(x.reshape(b * s, d), inds.reshape(b * s), pe)
    return out.reshape(b, s, d)
